# baseline (speedup 1.0000x reference)
import functools

import jax
import jax.numpy as jnp
from jax import lax
from jax.experimental import pallas as pl
from jax.experimental.pallas import tpu as pltpu
from jax.experimental.pallas import tpu_sc as plsc

CH = 16


def kernel(x, pe_weight):
    seq_len, batch, d_model = x.shape
    info = plsc.get_sparse_core_info()
    nw = info.num_cores * info.num_subcores
    rows_per_w = seq_len // nw
    n_chunks = rows_per_w // CH
    mesh = plsc.VectorSubcoreMesh(core_axis_name="c", subcore_axis_name="s")

    @functools.partial(
        pl.kernel,
        out_type=jax.ShapeDtypeStruct((seq_len, batch, d_model), jnp.float32),
        mesh=mesh,
        scratch_types=[
            pltpu.VMEM((CH, batch, d_model), jnp.float32),
            pltpu.VMEM((CH, d_model), jnp.float32),
        ],
    )
    def sc_add(x_hbm, pe_hbm, out_hbm, x_v, pe_v):
        wid = lax.axis_index("s") * info.num_cores + lax.axis_index("c")
        base = wid * rows_per_w

        @pl.loop(0, n_chunks)
        def _chunk(ci):
            row0 = base + ci * CH
            pltpu.sync_copy(x_hbm.at[pl.ds(row0, CH)], x_v)
            pltpu.sync_copy(pe_hbm.at[pl.ds(row0, CH)], pe_v)
            for r in range(CH):
                @pl.loop(0, d_model // 16, unroll=4)
                def _d(j):
                    dvec = pe_v[r, pl.ds(j * 16, 16)]
                    for b in range(batch):
                        x_v[r, b, pl.ds(j * 16, 16)] = (
                            x_v[r, b, pl.ds(j * 16, 16)] + dvec
                        )
            pltpu.sync_copy(x_v, out_hbm.at[pl.ds(row0, CH)])

    return sc_add(x, pe_weight)


# speedup vs baseline: 4.7594x; 4.7594x over previous
import jax
import jax.numpy as jnp
from jax import lax
from jax.experimental import pallas as pl
from jax.experimental.pallas import tpu as pltpu

BLK = 256
NIN = 4
NOUT = 3


def _body(x_hbm, pe_hbm, o_hbm, xb, pb, ob, sx, sp, so):
    n = pl.num_programs(0)
    i = pl.program_id(0)
    islot = lax.rem(i, NIN)
    oslot = lax.rem(i, NOUT)

    def in_copy(blk, slot):
        return (
            pltpu.make_async_copy(
                x_hbm.at[pl.ds(blk * BLK, BLK)], xb.at[slot], sx.at[slot]
            ),
            pltpu.make_async_copy(
                pe_hbm.at[pl.ds(blk * BLK, BLK)], pb.at[slot], sp.at[slot]
            ),
        )

    def out_copy(blk, slot):
        return pltpu.make_async_copy(
            ob.at[slot], o_hbm.at[pl.ds(blk * BLK, BLK)], so.at[slot]
        )

    @pl.when(i == 0)
    def _warmup():
        for k in range(NIN):
            cx, cp = in_copy(k, k)
            cx.start()
            cp.start()

    cx, cp = in_copy(i, islot)
    cx.wait()
    cp.wait()

    @pl.when(i >= NOUT)
    def _wait_out_slot():
        out_copy(i - NOUT, oslot).wait()

    ob[oslot] = xb[islot] + pb[islot][:, None, :]

    out_copy(i, oslot).start()

    @pl.when(i + NIN < n)
    def _refill():
        cx2, cp2 = in_copy(i + NIN, islot)
        cx2.start()
        cp2.start()

    @pl.when(i == n - 1)
    def _drain():
        for k in range(NOUT):
            out_copy(n - NOUT + k, lax.rem(n - NOUT + k, NOUT)).wait()


def kernel(x, pe_weight):
    seq_len, batch, d_model = x.shape
    n = seq_len // BLK
    return pl.pallas_call(
        _body,
        grid=(n,),
        in_specs=[
            pl.BlockSpec(memory_space=pltpu.HBM),
            pl.BlockSpec(memory_space=pltpu.HBM),
        ],
        out_specs=pl.BlockSpec(memory_space=pltpu.HBM),
        out_shape=jax.ShapeDtypeStruct((seq_len, batch, d_model), x.dtype),
        scratch_shapes=[
            pltpu.VMEM((NIN, BLK, batch, d_model), x.dtype),
            pltpu.VMEM((NIN, BLK, d_model), x.dtype),
            pltpu.VMEM((NOUT, BLK, batch, d_model), x.dtype),
            pltpu.SemaphoreType.DMA((NIN,)),
            pltpu.SemaphoreType.DMA((NIN,)),
            pltpu.SemaphoreType.DMA((NOUT,)),
        ],
        compiler_params=pltpu.CompilerParams(
            dimension_semantics=("arbitrary",),
        ),
    )(x, pe_weight)
